# baseline (speedup 1.0000x reference)
import functools

import jax
import jax.numpy as jnp
from jax import lax
from jax.experimental import pallas as pl
from jax.experimental.pallas import tpu as pltpu
from jax.experimental.pallas import tpu_sc as plsc

_CHUNK = 128
_WCOL = 16
_NC, _NS = 2, 16
_NW = _NC * _NS
_BR = 1024
_K = 4
_K2 = 8

_UNTILED = pltpu.CompilerParams(use_tc_tiling_on_sc=False)


def _sc_mesh():
    return plsc.VectorSubcoreMesh(core_axis_name="c", subcore_axis_name="s",
                                  num_cores=_NC, num_subcores=_NS)


def _deg_kernel(n_pad, cpw):
    rps = n_pad // _NS

    @functools.partial(
        pl.kernel,
        out_type=jax.ShapeDtypeStruct((_NC, n_pad, _WCOL), jnp.float32),
        mesh=_sc_mesh(),
        compiler_params=_UNTILED,
        scratch_types=(
            [pltpu.VMEM((cpw, _CHUNK), jnp.int32),
             pltpu.VMEM((_CHUNK, _WCOL), jnp.float32),
             pltpu.VMEM_SHARED((n_pad, _WCOL), jnp.float32)]
            + [pltpu.SemaphoreType.DMA] * _K2
        ),
    )
    def deg_kernel(dst_hbm, ones_hbm, zeros_hbm, out_hbm, dst_idx, ones_v, acc,
                   *sems):
        c = lax.axis_index("c")
        s = lax.axis_index("s")
        w = s * _NC + c
        stripe = pl.ds(s * rps, rps)
        pltpu.sync_copy(zeros_hbm.at[stripe], acc.at[stripe])
        pltpu.sync_copy(ones_hbm, ones_v)
        pltpu.sync_copy(dst_hbm.at[w], dst_idx)
        plsc.subcore_barrier()

        @pl.loop(0, cpw, step=_K2)
        def _(j):
            descs = [pltpu.async_copy(ones_v, acc.at[dst_idx.at[j + b]],
                                      sems[b], add=True) for b in range(_K2)]
            for dsc in descs:
                dsc.wait()

        plsc.subcore_barrier()
        pltpu.sync_copy(acc.at[stripe], out_hbm.at[c, stripe])

    return deg_kernel


def _pipelined_edge_loop(k, cpw, rows_src, src_idx, dst_idx, rows_v, acc,
                         gsem, ssem):
    for b in range(k):
        pltpu.async_copy(rows_src.at[src_idx.at[b]], rows_v.at[b], gsem[b])

    @pl.loop(0, cpw, step=k)
    def _(j):
        scat = []
        for b in range(k):
            pltpu.make_async_copy(rows_src.at[src_idx.at[j + b]],
                                  rows_v.at[b], gsem[b]).wait()
            scat.append(pltpu.async_copy(rows_v.at[b],
                                         acc.at[dst_idx.at[j + b]],
                                         ssem[b], add=True))
        for b in range(k):
            scat[b].wait()

            @pl.when(j + k + b < cpw)
            def _():
                pltpu.async_copy(rows_src.at[src_idx.at[j + k + b]],
                                 rows_v.at[b], gsem[b])


def _scatter1_kernel(n_pad, cpw, width):
    rps = n_pad // _NS
    hcw = cpw // 2

    @functools.partial(
        pl.kernel,
        out_type=jax.ShapeDtypeStruct((_NC, n_pad, width), jnp.float32),
        mesh=_sc_mesh(),
        scratch_types=(
            [pltpu.VMEM((hcw, _CHUNK), jnp.int32),
             pltpu.VMEM((hcw, _CHUNK), jnp.int32),
             pltpu.VMEM((2, _CHUNK, width), jnp.float32),
             pltpu.VMEM_SHARED((n_pad, width), jnp.float32)]
            + [pltpu.SemaphoreType.DMA] * 4
        ),
    )
    def scatter_kernel(rows_hbm, src_hbm, dst_hbm, zeros_hbm, out_hbm,
                       src_idx, dst_idx, rows_v, acc, *sems):
        gsem, ssem = sems[:2], sems[2:]
        c = lax.axis_index("c")
        s = lax.axis_index("s")
        w = s * _NC + c
        stripe = pl.ds(s * rps, rps)
        pltpu.sync_copy(zeros_hbm.at[stripe], acc.at[stripe])
        plsc.subcore_barrier()

        def run_half(j0):
            pltpu.sync_copy(src_hbm.at[w, pl.ds(j0, hcw)], src_idx)
            pltpu.sync_copy(dst_hbm.at[w, pl.ds(j0, hcw)], dst_idx)
            for b in range(2):
                pltpu.async_copy(rows_hbm.at[src_idx.at[b]], rows_v.at[b],
                                 gsem[b])

            @pl.loop(0, hcw, step=2)
            def _(i):
                scat = []
                for b in range(2):
                    pltpu.make_async_copy(rows_hbm.at[src_idx.at[i + b]],
                                          rows_v.at[b], gsem[b]).wait()
                    scat.append(pltpu.async_copy(rows_v.at[b],
                                                 acc.at[dst_idx.at[i + b]],
                                                 ssem[b], add=True))
                for b in range(2):
                    scat[b].wait()

                    @pl.when(i + 2 + b < hcw)
                    def _():
                        pltpu.async_copy(rows_hbm.at[src_idx.at[i + 2 + b]],
                                         rows_v.at[b], gsem[b])

        run_half(0)
        run_half(hcw)
        plsc.subcore_barrier()
        pltpu.sync_copy(acc.at[stripe], out_hbm.at[c, stripe])

    return scatter_kernel


def _scatter_kernel(n_pad, cpw, width):
    rps = n_pad // _NS

    @functools.partial(
        pl.kernel,
        out_type=jax.ShapeDtypeStruct((_NC, n_pad, width), jnp.float32),
        mesh=_sc_mesh(),
        compiler_params=_UNTILED,
        scratch_types=(
            [pltpu.VMEM((cpw, _CHUNK), jnp.int32),
             pltpu.VMEM((cpw, _CHUNK), jnp.int32),
             pltpu.VMEM((_K2, _CHUNK, width), jnp.float32),
             pltpu.VMEM_SHARED((n_pad, width), jnp.float32)]
            + [pltpu.SemaphoreType.DMA] * (2 * _K2)
        ),
    )
    def scatter_kernel(rows_hbm, src_hbm, dst_hbm, zeros_hbm, out_hbm,
                       src_idx, dst_idx, rows_v, acc, *sems):
        c = lax.axis_index("c")
        s = lax.axis_index("s")
        w = s * _NC + c
        stripe = pl.ds(s * rps, rps)
        pltpu.sync_copy(zeros_hbm.at[stripe], acc.at[stripe])
        pltpu.sync_copy(src_hbm.at[w], src_idx)
        pltpu.sync_copy(dst_hbm.at[w], dst_idx)
        plsc.subcore_barrier()
        _pipelined_edge_loop(_K2, cpw, rows_hbm, src_idx, dst_idx, rows_v,
                             acc, sems[:_K2], sems[_K2:])
        plsc.subcore_barrier()
        pltpu.sync_copy(acc.at[stripe], out_hbm.at[c, stripe])

    return scatter_kernel


def _dis_from(degp0, degp1):
    deg = degp0[:, 0:1] + degp1[:, 0:1] + 1.0
    return lax.rsqrt(deg)


def _mm_body(x_ref, w1_ref, h1_ref):
    h1_ref[...] = jnp.dot(x_ref[...], w1_ref[...],
                          preferred_element_type=jnp.float32)


def _scale_body(h1_ref, degp_ref, hs_ref):
    dis = _dis_from(degp_ref[0], degp_ref[1])
    hs_ref[...] = h1_ref[...] * dis


def _dense2_body(p_ref, hs_ref, degp_ref, b1_ref, w28_ref, ys8_ref):
    dis = _dis_from(degp_ref[0], degp_ref[1])
    acc = p_ref[0] + p_ref[1] + hs_ref[...]
    out1 = jnp.maximum(acc * dis + b1_ref[...], 0.0)
    ys8_ref[...] = jnp.dot(out1, w28_ref[...],
                           preferred_element_type=jnp.float32) * dis


def _final_body(p2_ref, ys8_ref, degp_ref, b28_ref, out_ref):
    dis = _dis_from(degp_ref[0], degp_ref[1])
    out_ref[...] = (p2_ref[0] + p2_ref[1] + ys8_ref[...]) * dis + b28_ref[...]


def kernel(x, edge_index, W1, b1, W2, b2):
    n, d = x.shape
    h = W1.shape[1]
    e = edge_index.shape[1]
    f32 = jnp.float32

    n_pad = ((n + _CHUNK) + _BR - 1) // _BR * _BR
    grid = n_pad // _BR
    grp = _CHUNK * _K2
    e_pad = (e + _NW * grp - 1) // (_NW * grp) * (_NW * grp)
    cpw = e_pad // _NW // _CHUNK

    x_p = jnp.zeros((n_pad, d), f32).at[:n].set(x)
    pad = n + (jnp.arange(e_pad - e, dtype=jnp.int32) % _CHUNK)
    src_f = jnp.concatenate([edge_index[0], pad])
    dst_f = jnp.concatenate([edge_index[1], pad])
    src32 = src_f.reshape(_NW, cpw, _CHUNK)
    dst32 = dst_f.reshape(_NW, cpw, _CHUNK)
    ones16 = jnp.ones((_CHUNK, _WCOL), f32)
    zeros16 = jnp.zeros((n_pad, _WCOL), f32)
    zerosd = jnp.zeros((n_pad, h), f32)
    w28 = jnp.broadcast_to(W2, (h, _WCOL)) if W2.shape[1] == 1 else W2
    b1r = b1.reshape(1, h)
    b28 = jnp.broadcast_to(b2.reshape(1, 1), (1, _WCOL))

    degp = _deg_kernel(n_pad, cpw)(dst32, ones16, zeros16)

    h1 = pl.pallas_call(
        _mm_body,
        grid=(grid,),
        in_specs=[
            pl.BlockSpec((_BR, d), lambda i: (i, 0)),
            pl.BlockSpec((d, h), lambda i: (0, 0)),
        ],
        out_specs=pl.BlockSpec((_BR, h), lambda i: (i, 0)),
        out_shape=jax.ShapeDtypeStruct((n_pad, h), f32),
    )(x_p, W1)

    hs = pl.pallas_call(
        _scale_body,
        grid=(grid,),
        in_specs=[
            pl.BlockSpec((_BR, h), lambda i: (i, 0)),
            pl.BlockSpec((_NC, _BR, _WCOL), lambda i: (0, i, 0)),
        ],
        out_specs=pl.BlockSpec((_BR, h), lambda i: (i, 0)),
        out_shape=jax.ShapeDtypeStruct((n_pad, h), f32),
    )(h1, degp)

    p1 = _scatter1_kernel(n_pad, cpw, h)(hs, src32, dst32, zerosd)

    ys8 = pl.pallas_call(
        _dense2_body,
        grid=(grid,),
        in_specs=[
            pl.BlockSpec((_NC, _BR, h), lambda i: (0, i, 0)),
            pl.BlockSpec((_BR, h), lambda i: (i, 0)),
            pl.BlockSpec((_NC, _BR, _WCOL), lambda i: (0, i, 0)),
            pl.BlockSpec((1, h), lambda i: (0, 0)),
            pl.BlockSpec((h, _WCOL), lambda i: (0, 0)),
        ],
        out_specs=pl.BlockSpec((_BR, _WCOL), lambda i: (i, 0)),
        out_shape=jax.ShapeDtypeStruct((n_pad, _WCOL), f32),
    )(p1, hs, degp, b1r, w28)

    p2 = _scatter_kernel(n_pad, cpw, _WCOL)(ys8, src32, dst32, zeros16)

    out8 = pl.pallas_call(
        _final_body,
        grid=(grid,),
        in_specs=[
            pl.BlockSpec((_NC, _BR, _WCOL), lambda i: (0, i, 0)),
            pl.BlockSpec((_BR, _WCOL), lambda i: (i, 0)),
            pl.BlockSpec((_NC, _BR, _WCOL), lambda i: (0, i, 0)),
            pl.BlockSpec((1, _WCOL), lambda i: (0, 0)),
        ],
        out_specs=pl.BlockSpec((_BR, _WCOL), lambda i: (i, 0)),
        out_shape=jax.ShapeDtypeStruct((n_pad, _WCOL), f32),
    )(p2, ys8, degp, b28)

    return out8[:n, 0]


# speedup vs baseline: 1.0665x; 1.0665x over previous
import functools

import jax
import jax.numpy as jnp
from jax import lax
from jax.experimental import pallas as pl
from jax.experimental.pallas import tpu as pltpu
from jax.experimental.pallas import tpu_sc as plsc

_CHUNK = 128
_WCOL = 16
_NC, _NS = 2, 16
_NW = _NC * _NS
_BR = 1024
_K = 4
_K2 = 8

_UNTILED = pltpu.CompilerParams(use_tc_tiling_on_sc=False)


def _sc_mesh():
    return plsc.VectorSubcoreMesh(core_axis_name="c", subcore_axis_name="s",
                                  num_cores=_NC, num_subcores=_NS)


def _deg_kernel(n_pad, cpw):
    rps = n_pad // _NS

    @functools.partial(
        pl.kernel,
        out_type=jax.ShapeDtypeStruct((_NC, n_pad, _WCOL), jnp.float32),
        mesh=_sc_mesh(),
        compiler_params=_UNTILED,
        scratch_types=(
            [pltpu.VMEM((cpw, _CHUNK), jnp.int32),
             pltpu.VMEM((_CHUNK, _WCOL), jnp.float32),
             pltpu.VMEM_SHARED((n_pad, _WCOL), jnp.float32)]
            + [pltpu.SemaphoreType.DMA] * _K2
        ),
    )
    def deg_kernel(dst_hbm, ones_hbm, zeros_hbm, out_hbm, dst_idx, ones_v, acc,
                   *sems):
        c = lax.axis_index("c")
        s = lax.axis_index("s")
        w = s * _NC + c
        stripe = pl.ds(s * rps, rps)
        pltpu.sync_copy(zeros_hbm.at[stripe], acc.at[stripe])
        pltpu.sync_copy(ones_hbm, ones_v)
        pltpu.sync_copy(dst_hbm.at[w], dst_idx)
        plsc.subcore_barrier()

        @pl.loop(0, cpw, step=_K2)
        def _(j):
            descs = [pltpu.async_copy(ones_v, acc.at[dst_idx.at[j + b]],
                                      sems[b], add=True) for b in range(_K2)]
            for dsc in descs:
                dsc.wait()

        plsc.subcore_barrier()
        pltpu.sync_copy(acc.at[stripe], out_hbm.at[c, stripe])

    return deg_kernel


def _pipelined_edge_loop(k, cpw, rows_src, src_idx, dst_idx, rows_v, acc,
                         gsem, ssem):
    for b in range(k):
        pltpu.async_copy(rows_src.at[src_idx.at[b]], rows_v.at[b], gsem[b])

    @pl.loop(0, cpw, step=k)
    def _(j):
        scat = []
        for b in range(k):
            pltpu.make_async_copy(rows_src.at[src_idx.at[j + b]],
                                  rows_v.at[b], gsem[b]).wait()
            scat.append(pltpu.async_copy(rows_v.at[b],
                                         acc.at[dst_idx.at[j + b]],
                                         ssem[b], add=True))
        for b in range(k):
            scat[b].wait()

            @pl.when(j + k + b < cpw)
            def _():
                pltpu.async_copy(rows_src.at[src_idx.at[j + k + b]],
                                 rows_v.at[b], gsem[b])


def _scatter_cols_kernel(n_pad, cps, width):
    rps = n_pad // _NS

    @functools.partial(
        pl.kernel,
        out_type=jax.ShapeDtypeStruct((_NC, n_pad, width), jnp.float32),
        mesh=_sc_mesh(),
        compiler_params=_UNTILED,
        scratch_types=(
            [pltpu.VMEM((cps, _CHUNK), jnp.int32),
             pltpu.VMEM((cps, _CHUNK), jnp.int32),
             pltpu.VMEM((_K, _CHUNK, width), jnp.float32),
             pltpu.VMEM_SHARED((n_pad, width), jnp.float32)]
            + [pltpu.SemaphoreType.DMA] * (2 * _K)
        ),
    )
    def scatter_kernel(rows_hbm, src_hbm, dst_hbm, zeros_hbm, out_hbm,
                       src_idx, dst_idx, rows_v, acc, *sems):
        c = lax.axis_index("c")
        s = lax.axis_index("s")
        stripe = pl.ds(s * rps, rps)
        pltpu.sync_copy(zeros_hbm.at[stripe], acc.at[stripe])
        pltpu.sync_copy(src_hbm.at[s], src_idx)
        pltpu.sync_copy(dst_hbm.at[s], dst_idx)
        plsc.subcore_barrier()
        _pipelined_edge_loop(_K, cps, rows_hbm.at[c], src_idx, dst_idx,
                             rows_v, acc, sems[:_K], sems[_K:])
        plsc.subcore_barrier()
        pltpu.sync_copy(acc.at[stripe], out_hbm.at[c, stripe])

    return scatter_kernel


def _scatter_kernel(n_pad, cpw, width):
    rps = n_pad // _NS

    @functools.partial(
        pl.kernel,
        out_type=jax.ShapeDtypeStruct((_NC, n_pad, width), jnp.float32),
        mesh=_sc_mesh(),
        compiler_params=_UNTILED,
        scratch_types=(
            [pltpu.VMEM((cpw, _CHUNK), jnp.int32),
             pltpu.VMEM((cpw, _CHUNK), jnp.int32),
             pltpu.VMEM((_K2, _CHUNK, width), jnp.float32),
             pltpu.VMEM_SHARED((n_pad, width), jnp.float32)]
            + [pltpu.SemaphoreType.DMA] * (2 * _K2)
        ),
    )
    def scatter_kernel(rows_hbm, src_hbm, dst_hbm, zeros_hbm, out_hbm,
                       src_idx, dst_idx, rows_v, acc, *sems):
        c = lax.axis_index("c")
        s = lax.axis_index("s")
        w = s * _NC + c
        stripe = pl.ds(s * rps, rps)
        pltpu.sync_copy(zeros_hbm.at[stripe], acc.at[stripe])
        pltpu.sync_copy(src_hbm.at[w], src_idx)
        pltpu.sync_copy(dst_hbm.at[w], dst_idx)
        plsc.subcore_barrier()
        _pipelined_edge_loop(_K2, cpw, rows_hbm, src_idx, dst_idx, rows_v,
                             acc, sems[:_K2], sems[_K2:])
        plsc.subcore_barrier()
        pltpu.sync_copy(acc.at[stripe], out_hbm.at[c, stripe])

    return scatter_kernel


def _dis_from(degp0, degp1):
    deg = degp0[:, 0:1] + degp1[:, 0:1] + 1.0
    return lax.rsqrt(deg)


def _mm_body(x_ref, w1_ref, h1_ref):
    h1_ref[...] = jnp.dot(x_ref[...], w1_ref[...],
                          preferred_element_type=jnp.float32)


def _scale_body(h1_ref, degp_ref, hs2_ref):
    dis = _dis_from(degp_ref[0], degp_ref[1])
    hs = h1_ref[...] * dis
    half = hs.shape[1] // _NC
    for c in range(_NC):
        hs2_ref[c] = hs[:, c * half:(c + 1) * half]


def _dense2_body(p_ref, hs2_ref, degp_ref, b1_ref, w28_ref, ys8_ref):
    dis = _dis_from(degp_ref[0], degp_ref[1])
    acc = jnp.concatenate([p_ref[0] + hs2_ref[0], p_ref[1] + hs2_ref[1]],
                          axis=1)
    out1 = jnp.maximum(acc * dis + b1_ref[...], 0.0)
    ys8_ref[...] = jnp.dot(out1, w28_ref[...],
                           preferred_element_type=jnp.float32) * dis


def _final_body(p2_ref, ys8_ref, degp_ref, b28_ref, out_ref):
    dis = _dis_from(degp_ref[0], degp_ref[1])
    out_ref[...] = (p2_ref[0] + p2_ref[1] + ys8_ref[...]) * dis + b28_ref[...]


def kernel(x, edge_index, W1, b1, W2, b2):
    n, d = x.shape
    h = W1.shape[1]
    e = edge_index.shape[1]
    f32 = jnp.float32
    half = h // _NC

    n_pad = ((n + _CHUNK) + _BR - 1) // _BR * _BR
    grid = n_pad // _BR
    grp = _CHUNK * _K2
    e_pad = (e + _NW * grp - 1) // (_NW * grp) * (_NW * grp)
    cps = e_pad // _NS // _CHUNK
    cpw = cps // _NC

    x_p = jnp.zeros((n_pad, d), f32).at[:n].set(x)
    pad = n + (jnp.arange(e_pad - e, dtype=jnp.int32) % _CHUNK)
    src_f = jnp.concatenate([edge_index[0], pad])
    dst_f = jnp.concatenate([edge_index[1], pad])
    src16 = src_f.reshape(_NS, cps, _CHUNK)
    dst16 = dst_f.reshape(_NS, cps, _CHUNK)
    src32 = src_f.reshape(_NW, cpw, _CHUNK)
    dst32 = dst_f.reshape(_NW, cpw, _CHUNK)
    ones16 = jnp.ones((_CHUNK, _WCOL), f32)
    zeros16 = jnp.zeros((n_pad, _WCOL), f32)
    zerosh = jnp.zeros((n_pad, half), f32)
    w28 = jnp.broadcast_to(W2, (h, _WCOL)) if W2.shape[1] == 1 else W2
    b1r = b1.reshape(1, h)
    b28 = jnp.broadcast_to(b2.reshape(1, 1), (1, _WCOL))

    degp = _deg_kernel(n_pad, cpw)(dst32, ones16, zeros16)

    h1 = pl.pallas_call(
        _mm_body,
        grid=(grid,),
        in_specs=[
            pl.BlockSpec((_BR, d), lambda i: (i, 0)),
            pl.BlockSpec((d, h), lambda i: (0, 0)),
        ],
        out_specs=pl.BlockSpec((_BR, h), lambda i: (i, 0)),
        out_shape=jax.ShapeDtypeStruct((n_pad, h), f32),
    )(x_p, W1)

    hs2 = pl.pallas_call(
        _scale_body,
        grid=(grid,),
        in_specs=[
            pl.BlockSpec((_BR, h), lambda i: (i, 0)),
            pl.BlockSpec((_NC, _BR, _WCOL), lambda i: (0, i, 0)),
        ],
        out_specs=pl.BlockSpec((_NC, _BR, half), lambda i: (0, i, 0)),
        out_shape=jax.ShapeDtypeStruct((_NC, n_pad, half), f32),
    )(h1, degp)

    p1 = _scatter_cols_kernel(n_pad, cps, half)(hs2, src16, dst16, zerosh)

    ys8 = pl.pallas_call(
        _dense2_body,
        grid=(grid,),
        in_specs=[
            pl.BlockSpec((_NC, _BR, half), lambda i: (0, i, 0)),
            pl.BlockSpec((_NC, _BR, half), lambda i: (0, i, 0)),
            pl.BlockSpec((_NC, _BR, _WCOL), lambda i: (0, i, 0)),
            pl.BlockSpec((1, h), lambda i: (0, 0)),
            pl.BlockSpec((h, _WCOL), lambda i: (0, 0)),
        ],
        out_specs=pl.BlockSpec((_BR, _WCOL), lambda i: (i, 0)),
        out_shape=jax.ShapeDtypeStruct((n_pad, _WCOL), f32),
    )(p1, hs2, degp, b1r, w28)

    p2 = _scatter_kernel(n_pad, cpw, _WCOL)(ys8, src32, dst32, zeros16)

    out8 = pl.pallas_call(
        _final_body,
        grid=(grid,),
        in_specs=[
            pl.BlockSpec((_NC, _BR, _WCOL), lambda i: (0, i, 0)),
            pl.BlockSpec((_BR, _WCOL), lambda i: (i, 0)),
            pl.BlockSpec((_NC, _BR, _WCOL), lambda i: (0, i, 0)),
            pl.BlockSpec((1, _WCOL), lambda i: (0, 0)),
        ],
        out_specs=pl.BlockSpec((_BR, _WCOL), lambda i: (i, 0)),
        out_shape=jax.ShapeDtypeStruct((n_pad, _WCOL), f32),
    )(p2, ys8, degp, b28)

    return out8[:n, 0]


# speedup vs baseline: 1.0743x; 1.0073x over previous
import functools

import jax
import jax.numpy as jnp
from jax import lax
from jax.experimental import pallas as pl
from jax.experimental.pallas import tpu as pltpu
from jax.experimental.pallas import tpu_sc as plsc

_CHUNK = 128
_WCOL = 16
_NC, _NS = 2, 16
_NW = _NC * _NS
_BR = 1024
_K = 5
_K2 = 8

_UNTILED = pltpu.CompilerParams(use_tc_tiling_on_sc=False)


def _sc_mesh():
    return plsc.VectorSubcoreMesh(core_axis_name="c", subcore_axis_name="s",
                                  num_cores=_NC, num_subcores=_NS)


def _deg_kernel(n_pad, cpw):
    rps = n_pad // _NS

    @functools.partial(
        pl.kernel,
        out_type=jax.ShapeDtypeStruct((_NC, n_pad, _WCOL), jnp.float32),
        mesh=_sc_mesh(),
        compiler_params=_UNTILED,
        scratch_types=(
            [pltpu.VMEM((cpw, _CHUNK), jnp.int32),
             pltpu.VMEM((_CHUNK, _WCOL), jnp.float32),
             pltpu.VMEM_SHARED((n_pad, _WCOL), jnp.float32)]
            + [pltpu.SemaphoreType.DMA] * _K2
        ),
    )
    def deg_kernel(dst_hbm, ones_hbm, zeros_hbm, out_hbm, dst_idx, ones_v, acc,
                   *sems):
        c = lax.axis_index("c")
        s = lax.axis_index("s")
        w = s * _NC + c
        stripe = pl.ds(s * rps, rps)
        pltpu.sync_copy(zeros_hbm.at[stripe], acc.at[stripe])
        pltpu.sync_copy(ones_hbm, ones_v)
        pltpu.sync_copy(dst_hbm.at[w], dst_idx)
        plsc.subcore_barrier()

        @pl.loop(0, cpw, step=_K2)
        def _(j):
            descs = [pltpu.async_copy(ones_v, acc.at[dst_idx.at[j + b]],
                                      sems[b], add=True) for b in range(_K2)]
            for dsc in descs:
                dsc.wait()

        plsc.subcore_barrier()
        pltpu.sync_copy(acc.at[stripe], out_hbm.at[c, stripe])

    return deg_kernel


def _pipelined_edge_loop(k, cpw, rows_src, src_idx, dst_idx, rows_v, acc,
                         gsem, ssem):
    for b in range(k):
        pltpu.async_copy(rows_src.at[src_idx.at[b]], rows_v.at[b], gsem[b])

    @pl.loop(0, cpw, step=k)
    def _(j):
        scat = []
        for b in range(k):
            pltpu.make_async_copy(rows_src.at[src_idx.at[j + b]],
                                  rows_v.at[b], gsem[b]).wait()
            scat.append(pltpu.async_copy(rows_v.at[b],
                                         acc.at[dst_idx.at[j + b]],
                                         ssem[b], add=True))
        for b in range(k):
            scat[b].wait()

            @pl.when(j + k + b < cpw)
            def _():
                pltpu.async_copy(rows_src.at[src_idx.at[j + k + b]],
                                 rows_v.at[b], gsem[b])


def _scatter_cols_kernel(n_pad, cps, width):
    rps = n_pad // _NS

    @functools.partial(
        pl.kernel,
        out_type=jax.ShapeDtypeStruct((_NC, n_pad, width), jnp.float32),
        mesh=_sc_mesh(),
        compiler_params=_UNTILED,
        scratch_types=(
            [pltpu.VMEM((cps, _CHUNK), jnp.int32),
             pltpu.VMEM((cps, _CHUNK), jnp.int32),
             pltpu.VMEM((_K, _CHUNK, width), jnp.float32),
             pltpu.VMEM_SHARED((n_pad, width), jnp.float32)]
            + [pltpu.SemaphoreType.DMA] * (2 * _K)
        ),
    )
    def scatter_kernel(rows_hbm, src_hbm, dst_hbm, zeros_hbm, out_hbm,
                       src_idx, dst_idx, rows_v, acc, *sems):
        c = lax.axis_index("c")
        s = lax.axis_index("s")
        stripe = pl.ds(s * rps, rps)
        pltpu.sync_copy(zeros_hbm.at[stripe], acc.at[stripe])
        pltpu.sync_copy(src_hbm.at[s], src_idx)
        pltpu.sync_copy(dst_hbm.at[s], dst_idx)
        plsc.subcore_barrier()
        _pipelined_edge_loop(_K, cps, rows_hbm.at[c], src_idx, dst_idx,
                             rows_v, acc, sems[:_K], sems[_K:])
        plsc.subcore_barrier()
        pltpu.sync_copy(acc.at[stripe], out_hbm.at[c, stripe])

    return scatter_kernel


def _scatter_kernel(n_pad, cpw, width):
    rps = n_pad // _NS

    @functools.partial(
        pl.kernel,
        out_type=jax.ShapeDtypeStruct((_NC, n_pad, width), jnp.float32),
        mesh=_sc_mesh(),
        compiler_params=_UNTILED,
        scratch_types=(
            [pltpu.VMEM((cpw, _CHUNK), jnp.int32),
             pltpu.VMEM((cpw, _CHUNK), jnp.int32),
             pltpu.VMEM((_K2, _CHUNK, width), jnp.float32),
             pltpu.VMEM_SHARED((n_pad, width), jnp.float32)]
            + [pltpu.SemaphoreType.DMA] * (2 * _K2)
        ),
    )
    def scatter_kernel(rows_hbm, src_hbm, dst_hbm, zeros_hbm, out_hbm,
                       src_idx, dst_idx, rows_v, acc, *sems):
        c = lax.axis_index("c")
        s = lax.axis_index("s")
        w = s * _NC + c
        stripe = pl.ds(s * rps, rps)
        pltpu.sync_copy(zeros_hbm.at[stripe], acc.at[stripe])
        pltpu.sync_copy(src_hbm.at[w], src_idx)
        pltpu.sync_copy(dst_hbm.at[w], dst_idx)
        plsc.subcore_barrier()
        _pipelined_edge_loop(_K2, cpw, rows_hbm, src_idx, dst_idx, rows_v,
                             acc, sems[:_K2], sems[_K2:])
        plsc.subcore_barrier()
        pltpu.sync_copy(acc.at[stripe], out_hbm.at[c, stripe])

    return scatter_kernel


def _dis_from(degp0, degp1):
    deg = degp0[:, 0:1] + degp1[:, 0:1] + 1.0
    return lax.rsqrt(deg)


def _mm_body(x_ref, w1_ref, h1_ref):
    h1_ref[...] = jnp.dot(x_ref[...], w1_ref[...],
                          preferred_element_type=jnp.float32)


def _scale_body(h1_ref, degp_ref, hs2_ref):
    dis = _dis_from(degp_ref[0], degp_ref[1])
    hs = h1_ref[...] * dis
    half = hs.shape[1] // _NC
    for c in range(_NC):
        hs2_ref[c] = hs[:, c * half:(c + 1) * half]


def _dense2_body(p_ref, hs2_ref, degp_ref, b1_ref, w28_ref, ys8_ref):
    dis = _dis_from(degp_ref[0], degp_ref[1])
    acc = jnp.concatenate([p_ref[0] + hs2_ref[0], p_ref[1] + hs2_ref[1]],
                          axis=1)
    out1 = jnp.maximum(acc * dis + b1_ref[...], 0.0)
    ys8_ref[...] = jnp.dot(out1, w28_ref[...],
                           preferred_element_type=jnp.float32) * dis


def _final_body(p2_ref, ys8_ref, degp_ref, b28_ref, out_ref):
    dis = _dis_from(degp_ref[0], degp_ref[1])
    out_ref[...] = (p2_ref[0] + p2_ref[1] + ys8_ref[...]) * dis + b28_ref[...]


def kernel(x, edge_index, W1, b1, W2, b2):
    n, d = x.shape
    h = W1.shape[1]
    e = edge_index.shape[1]
    f32 = jnp.float32
    half = h // _NC

    n_pad = ((n + _CHUNK) + _BR - 1) // _BR * _BR
    grid = n_pad // _BR
    grp = _CHUNK * _K2
    e_pad = (e + _NW * grp - 1) // (_NW * grp) * (_NW * grp)
    cps = e_pad // _NS // _CHUNK
    cpw = cps // _NC

    x_p = jnp.zeros((n_pad, d), f32).at[:n].set(x)
    pad = n + (jnp.arange(e_pad - e, dtype=jnp.int32) % _CHUNK)
    src_f = jnp.concatenate([edge_index[0], pad])
    dst_f = jnp.concatenate([edge_index[1], pad])
    src16 = src_f.reshape(_NS, cps, _CHUNK)
    dst16 = dst_f.reshape(_NS, cps, _CHUNK)
    src32 = src_f.reshape(_NW, cpw, _CHUNK)
    dst32 = dst_f.reshape(_NW, cpw, _CHUNK)
    ones16 = jnp.ones((_CHUNK, _WCOL), f32)
    zeros16 = jnp.zeros((n_pad, _WCOL), f32)
    zerosh = jnp.zeros((n_pad, half), f32)
    w28 = jnp.broadcast_to(W2, (h, _WCOL)) if W2.shape[1] == 1 else W2
    b1r = b1.reshape(1, h)
    b28 = jnp.broadcast_to(b2.reshape(1, 1), (1, _WCOL))

    degp = _deg_kernel(n_pad, cpw)(dst32, ones16, zeros16)

    h1 = pl.pallas_call(
        _mm_body,
        grid=(grid,),
        in_specs=[
            pl.BlockSpec((_BR, d), lambda i: (i, 0)),
            pl.BlockSpec((d, h), lambda i: (0, 0)),
        ],
        out_specs=pl.BlockSpec((_BR, h), lambda i: (i, 0)),
        out_shape=jax.ShapeDtypeStruct((n_pad, h), f32),
    )(x_p, W1)

    hs2 = pl.pallas_call(
        _scale_body,
        grid=(grid,),
        in_specs=[
            pl.BlockSpec((_BR, h), lambda i: (i, 0)),
            pl.BlockSpec((_NC, _BR, _WCOL), lambda i: (0, i, 0)),
        ],
        out_specs=pl.BlockSpec((_NC, _BR, half), lambda i: (0, i, 0)),
        out_shape=jax.ShapeDtypeStruct((_NC, n_pad, half), f32),
    )(h1, degp)

    p1 = _scatter_cols_kernel(n_pad, cps, half)(hs2, src16, dst16, zerosh)

    ys8 = pl.pallas_call(
        _dense2_body,
        grid=(grid,),
        in_specs=[
            pl.BlockSpec((_NC, _BR, half), lambda i: (0, i, 0)),
            pl.BlockSpec((_NC, _BR, half), lambda i: (0, i, 0)),
            pl.BlockSpec((_NC, _BR, _WCOL), lambda i: (0, i, 0)),
            pl.BlockSpec((1, h), lambda i: (0, 0)),
            pl.BlockSpec((h, _WCOL), lambda i: (0, 0)),
        ],
        out_specs=pl.BlockSpec((_BR, _WCOL), lambda i: (i, 0)),
        out_shape=jax.ShapeDtypeStruct((n_pad, _WCOL), f32),
    )(p1, hs2, degp, b1r, w28)

    p2 = _scatter_kernel(n_pad, cpw, _WCOL)(ys8, src32, dst32, zeros16)

    out8 = pl.pallas_call(
        _final_body,
        grid=(grid,),
        in_specs=[
            pl.BlockSpec((_NC, _BR, _WCOL), lambda i: (0, i, 0)),
            pl.BlockSpec((_BR, _WCOL), lambda i: (i, 0)),
            pl.BlockSpec((_NC, _BR, _WCOL), lambda i: (0, i, 0)),
            pl.BlockSpec((1, _WCOL), lambda i: (0, 0)),
        ],
        out_specs=pl.BlockSpec((_BR, _WCOL), lambda i: (i, 0)),
        out_shape=jax.ShapeDtypeStruct((n_pad, _WCOL), f32),
    )(p2, ys8, degp, b28)

    return out8[:n, 0]
